# Initial kernel scaffold; baseline (speedup 1.0000x reference)
import jax
import jax.numpy as jnp
from jax.experimental import pallas as pl


def kernel(x, W1, b1, W2, b2, W3, b3, embeddings):
    raise NotImplementedError("write your pallas kernel here")


# speedup vs baseline: 1.3414x; 1.3414x over previous
import functools

import jax
import jax.numpy as jnp
from jax import lax
from jax.experimental import pallas as pl
from jax.experimental.pallas import tpu as pltpu
from jax.experimental.pallas import tpu_sc as plsc

INPUT_SIZE = 512
HIDDEN = 1024
EMBED_DIM = 256
NUM_EMB = 8192
BATCH = 16384

ROW_TILE = 256
GRID = BATCH // ROW_TILE


def _encode_body(x_ref, w1_ref, b1_ref, w2_ref, b2_ref, w3_ref, b3_ref,
                 emb_ref, enorm_ref, idx_ref):
    x = x_ref[...]
    h1 = jax.nn.relu(jnp.dot(x, w1_ref[...]) + b1_ref[...])
    h2 = jax.nn.relu(jnp.dot(h1, w2_ref[...]) + b2_ref[...])
    f = jnp.dot(h2, w3_ref[...]) + b3_ref[...]
    fnorm = jnp.sum(f ** 2, axis=1, keepdims=True)
    m = lax.dot_general(f, emb_ref[...], (((1,), (1,)), ((), ())))
    d = (fnorm + enorm_ref[...]) - 2.0 * m
    dmin = jnp.min(d, axis=1, keepdims=True)
    iota = lax.broadcasted_iota(jnp.int32, d.shape, 1)
    idx = jnp.min(jnp.where(d == dmin, iota, NUM_EMB), axis=1)
    idx_ref[...] = idx.reshape(1, 1, ROW_TILE)


def _encode_indices(x, W1, b1, W2, b2, W3, b3, embeddings, enorm):
    return pl.pallas_call(
        _encode_body,
        grid=(GRID,),
        in_specs=[
            pl.BlockSpec((ROW_TILE, INPUT_SIZE), lambda i: (i, 0)),
            pl.BlockSpec((INPUT_SIZE, HIDDEN), lambda i: (0, 0)),
            pl.BlockSpec((1, HIDDEN), lambda i: (0, 0)),
            pl.BlockSpec((HIDDEN, HIDDEN), lambda i: (0, 0)),
            pl.BlockSpec((1, HIDDEN), lambda i: (0, 0)),
            pl.BlockSpec((HIDDEN, EMBED_DIM), lambda i: (0, 0)),
            pl.BlockSpec((1, EMBED_DIM), lambda i: (0, 0)),
            pl.BlockSpec((NUM_EMB, EMBED_DIM), lambda i: (0, 0)),
            pl.BlockSpec((1, NUM_EMB), lambda i: (0, 0)),
        ],
        out_specs=pl.BlockSpec((1, 1, ROW_TILE), lambda i: (i, 0, 0)),
        out_shape=jax.ShapeDtypeStruct((GRID, 1, ROW_TILE), jnp.int32),
    )(x, W1, b1.reshape(1, HIDDEN), W2, b2.reshape(1, HIDDEN),
      W3, b3.reshape(1, EMBED_DIM), embeddings, enorm.reshape(1, NUM_EMB))


_SC_INFO = plsc.get_sparse_core_info()
_NC = _SC_INFO.num_cores
_NS = _SC_INFO.num_subcores
_NW = _NC * _NS
_BPW = BATCH // _NW
_CHUNK = 128
_NCHUNK = _BPW // _CHUNK


def _gather_rows(embeddings, idx):
    mesh = plsc.VectorSubcoreMesh(core_axis_name="c", subcore_axis_name="s")

    @functools.partial(
        pl.kernel,
        mesh=mesh,
        out_type=jax.ShapeDtypeStruct((BATCH, EMBED_DIM), jnp.float32),
        scratch_types=[
            pltpu.VMEM((_CHUNK,), jnp.int32),
            pltpu.VMEM((_CHUNK, EMBED_DIM), jnp.float32),
            pltpu.SemaphoreType.DMA,
        ],
    )
    def k(table_hbm, idx_hbm, out_hbm, idx_v, rows_v, sem):
        wid = lax.axis_index("s") * _NC + lax.axis_index("c")

        def body(c, carry):
            base = wid * _BPW + c * _CHUNK
            pltpu.sync_copy(idx_hbm.at[pl.ds(base, _CHUNK)], idx_v)
            pltpu.async_copy(table_hbm.at[idx_v], rows_v, sem).wait()
            pltpu.sync_copy(rows_v, out_hbm.at[pl.ds(base, _CHUNK)])
            return carry

        lax.fori_loop(0, _NCHUNK, body, 0)

    return k(embeddings, idx)


def kernel(x, W1, b1, W2, b2, W3, b3, embeddings):
    enorm = jnp.sum(embeddings ** 2, axis=1)
    idx = _encode_indices(x, W1, b1, W2, b2, W3, b3, embeddings, enorm)
    idx = idx.reshape(BATCH)
    return _gather_rows(embeddings, idx)


# speedup vs baseline: 1.3426x; 1.0010x over previous
import functools

import jax
import jax.numpy as jnp
from jax import lax
from jax.experimental import pallas as pl
from jax.experimental.pallas import tpu as pltpu
from jax.experimental.pallas import tpu_sc as plsc

INPUT_SIZE = 512
HIDDEN = 1024
EMBED_DIM = 256
NUM_EMB = 8192
BATCH = 16384

ROW_TILE = 512
GRID = BATCH // ROW_TILE
EMB_CHUNK = 2048
N_CHUNKS = NUM_EMB // EMB_CHUNK


def _encode_body(x_ref, w1_ref, b1_ref, w2_ref, b2_ref, w3_ref, b3_ref,
                 emb_ref, enorm_ref, idx_ref):
    x = x_ref[...]
    h1 = jax.nn.relu(jnp.dot(x, w1_ref[...]) + b1_ref[...])
    h2 = jax.nn.relu(jnp.dot(h1, w2_ref[...]) + b2_ref[...])
    f = jnp.dot(h2, w3_ref[...]) + b3_ref[...]
    fnorm = jnp.sum(f ** 2, axis=1, keepdims=True)
    best_val = None
    best_idx = None
    for c in range(N_CHUNKS):
        emb_c = emb_ref[c * EMB_CHUNK:(c + 1) * EMB_CHUNK, :]
        m = lax.dot_general(f, emb_c, (((1,), (1,)), ((), ())))
        enorm_c = enorm_ref[:, c * EMB_CHUNK:(c + 1) * EMB_CHUNK]
        d = (fnorm + enorm_c) - 2.0 * m
        dmin = jnp.min(d, axis=1, keepdims=True)
        iota = lax.broadcasted_iota(jnp.int32, d.shape, 1) + c * EMB_CHUNK
        imin = jnp.min(jnp.where(d == dmin, iota, NUM_EMB), axis=1, keepdims=True)
        if best_val is None:
            best_val, best_idx = dmin, imin
        else:
            better = dmin < best_val
            best_idx = jnp.where(better, imin, best_idx)
            best_val = jnp.where(better, dmin, best_val)
    idx_ref[...] = best_idx.reshape(1, 1, ROW_TILE)


def _encode_indices(x, W1, b1, W2, b2, W3, b3, embeddings, enorm):
    return pl.pallas_call(
        _encode_body,
        grid=(GRID,),
        in_specs=[
            pl.BlockSpec((ROW_TILE, INPUT_SIZE), lambda i: (i, 0)),
            pl.BlockSpec((INPUT_SIZE, HIDDEN), lambda i: (0, 0)),
            pl.BlockSpec((1, HIDDEN), lambda i: (0, 0)),
            pl.BlockSpec((HIDDEN, HIDDEN), lambda i: (0, 0)),
            pl.BlockSpec((1, HIDDEN), lambda i: (0, 0)),
            pl.BlockSpec((HIDDEN, EMBED_DIM), lambda i: (0, 0)),
            pl.BlockSpec((1, EMBED_DIM), lambda i: (0, 0)),
            pl.BlockSpec((NUM_EMB, EMBED_DIM), lambda i: (0, 0)),
            pl.BlockSpec((1, NUM_EMB), lambda i: (0, 0)),
        ],
        out_specs=pl.BlockSpec((1, 1, ROW_TILE), lambda i: (i, 0, 0)),
        out_shape=jax.ShapeDtypeStruct((GRID, 1, ROW_TILE), jnp.int32),
    )(x, W1, b1.reshape(1, HIDDEN), W2, b2.reshape(1, HIDDEN),
      W3, b3.reshape(1, EMBED_DIM), embeddings, enorm.reshape(1, NUM_EMB))


_SC_INFO = plsc.get_sparse_core_info()
_NC = _SC_INFO.num_cores
_NS = _SC_INFO.num_subcores
_NW = _NC * _NS
_BPW = BATCH // _NW
_CHUNK = 128
_NCHUNK = _BPW // _CHUNK


def _gather_rows(embeddings, idx):
    mesh = plsc.VectorSubcoreMesh(core_axis_name="c", subcore_axis_name="s")

    @functools.partial(
        pl.kernel,
        mesh=mesh,
        out_type=jax.ShapeDtypeStruct((BATCH, EMBED_DIM), jnp.float32),
        scratch_types=[
            pltpu.VMEM((_CHUNK,), jnp.int32),
            pltpu.VMEM((_CHUNK, EMBED_DIM), jnp.float32),
            pltpu.SemaphoreType.DMA,
        ],
    )
    def k(table_hbm, idx_hbm, out_hbm, idx_v, rows_v, sem):
        wid = lax.axis_index("s") * _NC + lax.axis_index("c")

        def body(c, carry):
            base = wid * _BPW + c * _CHUNK
            pltpu.sync_copy(idx_hbm.at[pl.ds(base, _CHUNK)], idx_v)
            pltpu.async_copy(table_hbm.at[idx_v], rows_v, sem).wait()
            pltpu.sync_copy(rows_v, out_hbm.at[pl.ds(base, _CHUNK)])
            return carry

        lax.fori_loop(0, _NCHUNK, body, 0)

    return k(embeddings, idx)


def kernel(x, W1, b1, W2, b2, W3, b3, embeddings):
    enorm = jnp.sum(embeddings ** 2, axis=1)
    idx = _encode_indices(x, W1, b1, W2, b2, W3, b3, embeddings, enorm)
    idx = idx.reshape(BATCH)
    return _gather_rows(embeddings, idx)
